# Initial kernel scaffold; baseline (speedup 1.0000x reference)
import jax
import jax.numpy as jnp
from jax.experimental import pallas as pl


def kernel(input_seq, embedding_weight):
    raise NotImplementedError("write your pallas kernel here")


# speedup vs baseline: 5.5489x; 5.5489x over previous
import functools

import jax
import jax.numpy as jnp
from jax import lax
from jax.experimental import pallas as pl
from jax.experimental.pallas import tpu as pltpu
from jax.experimental.pallas import tpu_sc as plsc

VOCAB = 1000
D = 128
BATCH = 4096
SEQ = 200
N = BATCH * SEQ

NC = 2
NS = 16
NW = NC * NS
RPW = N // NW
CHUNK = 128
CHUNKS = RPW // CHUNK


@functools.partial(
    pl.kernel,
    out_type=jax.ShapeDtypeStruct((N, D), jnp.float32),
    mesh=plsc.VectorSubcoreMesh(core_axis_name="c", subcore_axis_name="s"),
    scratch_types=[
        pltpu.VMEM((CHUNKS, CHUNK), jnp.int32),
        pltpu.VMEM((CHUNK, D), jnp.float32),
        pltpu.SemaphoreType.DMA,
    ],
)
def _emb_lookup(table_hbm, idx_hbm, out_hbm, idx_v, rows_v, sem):
    wid = lax.axis_index("s") * NC + lax.axis_index("c")
    pltpu.sync_copy(idx_hbm.at[wid], idx_v)

    def body(j, carry):
        pltpu.async_copy(table_hbm.at[idx_v.at[j]], rows_v, sem).wait()
        pltpu.sync_copy(rows_v, out_hbm.at[pl.ds(wid * RPW + j * CHUNK, CHUNK)])
        return carry

    lax.fori_loop(0, CHUNKS, body, 0)


def kernel(input_seq, embedding_weight):
    idx = input_seq.reshape(NW, CHUNKS, CHUNK).astype(jnp.int32)
    out = _emb_lookup(embedding_weight, idx)
    return out.reshape(BATCH, SEQ, D)


# speedup vs baseline: 6.5461x; 1.1797x over previous
import functools

import jax
import jax.numpy as jnp
from jax import lax
from jax.experimental import pallas as pl
from jax.experimental.pallas import tpu as pltpu
from jax.experimental.pallas import tpu_sc as plsc

VOCAB = 1000
D = 128
BATCH = 4096
SEQ = 200
N = BATCH * SEQ

NC = 2
NS = 16
NW = NC * NS
RPW = N // NW
CHUNK = 128
CHUNKS = RPW // CHUNK
NBUF = 4
GROUPS = CHUNKS // NBUF


@functools.partial(
    pl.kernel,
    out_type=jax.ShapeDtypeStruct((N, D), jnp.float32),
    mesh=plsc.VectorSubcoreMesh(core_axis_name="c", subcore_axis_name="s"),
    scratch_types=[
        pltpu.VMEM((CHUNKS, CHUNK), jnp.int32),
        pltpu.VMEM((NBUF, CHUNK, D), jnp.float32),
        pltpu.SemaphoreType.DMA((NBUF,)),
        pltpu.SemaphoreType.DMA((NBUF,)),
    ],
)
def _emb_lookup(table_hbm, idx_hbm, out_hbm, idx_v, rows_v, gsem, ssem):
    wid = lax.axis_index("s") * NC + lax.axis_index("c")
    pltpu.sync_copy(idx_hbm.at[wid], idx_v)
    out_base = wid * RPW

    def gather(j, b):
        pltpu.async_copy(table_hbm.at[idx_v.at[j]], rows_v.at[b], gsem.at[b])

    def store(j, b):
        pltpu.async_copy(
            rows_v.at[b], out_hbm.at[pl.ds(out_base + j * CHUNK, CHUNK)],
            ssem.at[b])

    def wait(sem, b):
        pltpu.make_async_copy(
            table_hbm.at[pl.ds(0, CHUNK)], rows_v.at[b], sem.at[b]).wait()

    for b in range(NBUF):
        gather(b, b)

    def body(i, carry):
        for b in range(NBUF):
            j = i * NBUF + b
            wait(gsem, b)
            store(j, b)
            wait(ssem, b)
            gather(j + NBUF, b)
        return carry

    lax.fori_loop(0, GROUPS - 1, body, 0)

    for b in range(NBUF):
        j = (GROUPS - 1) * NBUF + b
        wait(gsem, b)
        store(j, b)
    for b in range(NBUF):
        wait(ssem, b)


def kernel(input_seq, embedding_weight):
    idx = input_seq.reshape(NW, CHUNKS, CHUNK).astype(jnp.int32)
    out = _emb_lookup(embedding_weight, idx)
    return out.reshape(BATCH, SEQ, D)


# speedup vs baseline: 15.9472x; 2.4362x over previous
import functools

import jax
import jax.numpy as jnp
from jax import lax
from jax.experimental import pallas as pl
from jax.experimental.pallas import tpu as pltpu
from jax.experimental.pallas import tpu_sc as plsc

VOCAB = 1000
D = 128
BATCH = 4096
SEQ = 200
N = BATCH * SEQ

NC = 2
NS = 16
NW = NC * NS
RPW = N // NW
CHUNK = 128
CHUNKS = RPW // CHUNK
NBUF = 4
GROUPS = CHUNKS // NBUF
STAGERS = 5
VPS = VOCAB // STAGERS


@functools.partial(
    pl.kernel,
    out_type=jax.ShapeDtypeStruct((N, D), jnp.float32),
    mesh=plsc.VectorSubcoreMesh(core_axis_name="c", subcore_axis_name="s"),
    scratch_types=[
        pltpu.VMEM((CHUNKS, CHUNK), jnp.int32),
        pltpu.VMEM((NBUF, CHUNK, D), jnp.float32),
        pltpu.VMEM((VPS, D), jnp.float32),
        pltpu.VMEM_SHARED((VOCAB, D), jnp.float32),
        pltpu.SemaphoreType.DMA((NBUF,)),
        pltpu.SemaphoreType.DMA((NBUF,)),
    ],
)
def _emb_lookup(table_hbm, idx_hbm, out_hbm, idx_v, rows_v, stage_v,
                table_spm, gsem, ssem):
    sid = lax.axis_index("s")
    wid = sid * NC + lax.axis_index("c")

    @pl.when(sid < STAGERS)
    def _stage():
        pltpu.sync_copy(table_hbm.at[pl.ds(sid * VPS, VPS)], stage_v)
        pltpu.sync_copy(stage_v, table_spm.at[pl.ds(sid * VPS, VPS)])

    pltpu.sync_copy(idx_hbm.at[wid], idx_v)
    plsc.subcore_barrier()
    out_base = wid * RPW

    def gather(j, b):
        pltpu.async_copy(table_spm.at[idx_v.at[j]], rows_v.at[b], gsem.at[b])

    def store(j, b):
        pltpu.async_copy(
            rows_v.at[b], out_hbm.at[pl.ds(out_base + j * CHUNK, CHUNK)],
            ssem.at[b])

    def wait(sem, b):
        pltpu.make_async_copy(
            table_hbm.at[pl.ds(0, CHUNK)], rows_v.at[b], sem.at[b]).wait()

    for b in range(NBUF):
        gather(b, b)

    def body(i, carry):
        for b in range(NBUF):
            j = i * NBUF + b
            wait(gsem, b)
            store(j, b)
            wait(ssem, b)
            gather(j + NBUF, b)
        return carry

    lax.fori_loop(0, GROUPS - 1, body, 0)

    for b in range(NBUF):
        j = (GROUPS - 1) * NBUF + b
        wait(gsem, b)
        store(j, b)
    for b in range(NBUF):
        wait(ssem, b)


def kernel(input_seq, embedding_weight):
    idx = input_seq.reshape(NW, CHUNKS, CHUNK).astype(jnp.int32)
    out = _emb_lookup(embedding_weight, idx)
    return out.reshape(BATCH, SEQ, D)
